# baseline (speedup 1.0000x reference)
import math

import jax
import jax.numpy as jnp
from jax.experimental import pallas as pl
from jax.experimental.pallas import tpu as pltpu

S = 64.0
M = 0.5
EPS = 1e-07
COS_M = math.cos(M)
SIN_M = math.sin(M)
EXPC = S * math.log2(math.e)

NROW = 200
NBUF = 7


def _loss_body(lab_ref, wf_hbm, o_ref, bufs, sems, acc_e_ref, acc_t_ref):
    c, nb = wf_hbm.shape
    nstep = c // NROW

    def _chunk_copy(step, slot):
        return pltpu.make_async_copy(
            wf_hbm.at[pl.ds(step * NROW, NROW), :],
            bufs.at[slot],
            sems.at[slot],
        )

    for s in range(NBUF):
        _chunk_copy(s, s).start()

    acc_e_ref[...] = jnp.zeros_like(acc_e_ref)
    acc_t_ref[...] = jnp.zeros_like(acc_t_ref)
    lab = lab_ref[...]
    iota8 = jax.lax.broadcasted_iota(jnp.int32, (8, nb), 0)
    rows8 = [iota8 + r * 8 for r in range(NROW // 8)]

    def _step(i, carry):
        slot = jax.lax.rem(i, NBUF)
        pltpu.make_async_copy(bufs.at[slot], bufs.at[slot],
                              sems.at[slot]).wait()
        blk_all = bufs.at[slot]
        lab_rel = lab - i * NROW
        loc_e = jnp.zeros((8, nb), jnp.float32)
        loc_t = jnp.zeros((8, nb), jnp.float32)
        for r in range(NROW // 8):
            blk = blk_all[r * 8:(r + 1) * 8, :]
            hit = rows8[r] == lab_rel
            loc_e = loc_e + jnp.exp2(blk * EXPC)
            loc_t = loc_t + jnp.where(hit, blk, 0.0)
        acc_e_ref[...] = acc_e_ref[...] + loc_e
        acc_t_ref[...] = acc_t_ref[...] + loc_t

        @pl.when(i + NBUF < nstep)
        def _():
            _chunk_copy(i + NBUF, slot).start()

        return carry

    jax.lax.fori_loop(0, nstep, _step, 0)

    rowsum = jnp.sum(acc_e_ref[...], axis=0, keepdims=True)
    tgt = jnp.sum(acc_t_ref[...], axis=0, keepdims=True)
    t = jnp.clip(tgt, -1.0 + EPS, 1.0 - EPS)
    num = S * (t * COS_M - jnp.sqrt(1.0 - t * t) * SIN_M)
    den = jnp.exp(num) + (rowsum - jnp.exp(S * tgt))
    loss = num - jnp.log(den)
    o_ref[0, 0] = jnp.sum(loss) * (-1.0 / nb)


def kernel(wf, labels):
    b, c = wf.shape
    wft = wf.T
    lab2 = labels.astype(jnp.int32).reshape(1, b)
    out = pl.pallas_call(
        _loss_body,
        in_specs=[
            pl.BlockSpec(memory_space=pltpu.VMEM),
            pl.BlockSpec(memory_space=pl.ANY),
        ],
        out_specs=pl.BlockSpec(memory_space=pltpu.SMEM),
        out_shape=jax.ShapeDtypeStruct((1, 1), jnp.float32),
        scratch_shapes=[
            pltpu.VMEM((NBUF, NROW, b), jnp.float32),
            pltpu.SemaphoreType.DMA((NBUF,)),
            pltpu.VMEM((8, b), jnp.float32),
            pltpu.VMEM((8, b), jnp.float32),
        ],
        compiler_params=pltpu.CompilerParams(
            vmem_limit_bytes=52 * 1024 * 1024,
        ),
        name="arcface_loss",
    )(lab2, wft)
    return out.reshape(())


# speedup vs baseline: 1.0081x; 1.0081x over previous
import math

import jax
import jax.numpy as jnp
from jax.experimental import pallas as pl
from jax.experimental.pallas import tpu as pltpu

S = 64.0
M = 0.5
EPS = 1e-07
COS_M = math.cos(M)
SIN_M = math.sin(M)
EXPC = S * math.log2(math.e)

NROW = 80
NBUF = 10


def _loss_body(lab_ref, wf_hbm, o_ref, bufs, sems, acc_e_ref, acc_t_ref):
    c, nb = wf_hbm.shape
    nstep = c // NROW

    def _chunk_copy(step, slot):
        return pltpu.make_async_copy(
            wf_hbm.at[pl.ds(step * NROW, NROW), :],
            bufs.at[slot],
            sems.at[slot],
        )

    for s in range(NBUF):
        _chunk_copy(s, s).start()

    acc_e_ref[...] = jnp.zeros_like(acc_e_ref)
    acc_t_ref[...] = jnp.zeros_like(acc_t_ref)
    lab = lab_ref[...]
    iota8 = jax.lax.broadcasted_iota(jnp.int32, (8, nb), 0)
    rows8 = [iota8 + r * 8 for r in range(NROW // 8)]

    def _step(i, carry):
        slot = jax.lax.rem(i, NBUF)
        pltpu.make_async_copy(bufs.at[slot], bufs.at[slot],
                              sems.at[slot]).wait()
        blk_all = bufs.at[slot]
        lab_rel = lab - i * NROW
        loc_e = jnp.zeros((8, nb), jnp.float32)
        loc_t = jnp.zeros((8, nb), jnp.float32)
        for r in range(NROW // 8):
            blk = blk_all[r * 8:(r + 1) * 8, :]
            hit = rows8[r] == lab_rel
            loc_e = loc_e + jnp.exp2(blk * EXPC)
            loc_t = loc_t + jnp.where(hit, blk, 0.0)
        acc_e_ref[...] = acc_e_ref[...] + loc_e
        acc_t_ref[...] = acc_t_ref[...] + loc_t

        @pl.when(i + NBUF < nstep)
        def _():
            _chunk_copy(i + NBUF, slot).start()

        return carry

    jax.lax.fori_loop(0, nstep, _step, 0)

    rowsum = jnp.sum(acc_e_ref[...], axis=0, keepdims=True)
    tgt = jnp.sum(acc_t_ref[...], axis=0, keepdims=True)
    t = jnp.clip(tgt, -1.0 + EPS, 1.0 - EPS)
    num = S * (t * COS_M - jnp.sqrt(1.0 - t * t) * SIN_M)
    den = jnp.exp(num) + (rowsum - jnp.exp(S * tgt))
    loss = num - jnp.log(den)
    o_ref[0, 0] = jnp.sum(loss) * (-1.0 / nb)


def kernel(wf, labels):
    b, c = wf.shape
    wft = wf.T
    lab2 = labels.astype(jnp.int32).reshape(1, b)
    out = pl.pallas_call(
        _loss_body,
        in_specs=[
            pl.BlockSpec(memory_space=pltpu.VMEM),
            pl.BlockSpec(memory_space=pl.ANY),
        ],
        out_specs=pl.BlockSpec(memory_space=pltpu.SMEM),
        out_shape=jax.ShapeDtypeStruct((1, 1), jnp.float32),
        scratch_shapes=[
            pltpu.VMEM((NBUF, NROW, b), jnp.float32),
            pltpu.SemaphoreType.DMA((NBUF,)),
            pltpu.VMEM((8, b), jnp.float32),
            pltpu.VMEM((8, b), jnp.float32),
        ],
        compiler_params=pltpu.CompilerParams(
            vmem_limit_bytes=52 * 1024 * 1024,
        ),
        name="arcface_loss",
    )(lab2, wft)
    return out.reshape(())
